# baseline (speedup 1.0000x reference)
import functools

import jax
import jax.numpy as jnp
from jax import lax
from jax.experimental import pallas as pl

NUM_NODES = 10000
TIME_STEPS = 64
LATENT_DIM = 256
NUM_CLASSES = [64, 8]
TEMPERATURE = 0.5
BATCH = 16

NODE_BLK = 1000
N_NODE_BLKS = NUM_NODES // NODE_BLK


def _emb_var_body(x_ref, w_ref, b_ref, emb_ref, var_ref):
    xb = x_ref[0]
    w = w_ref[...]
    emb = lax.dot_general(xb, w, (((1,), (1,)), ((), ())),
                          preferred_element_type=jnp.float32)
    emb = emb + b_ref[...]
    emb_ref[0] = emb
    s = jnp.sum(emb, axis=1)
    ss = jnp.sum(emb * emb, axis=1)
    n = jnp.float32(LATENT_DIM)
    var = (ss - s * s / n) / (n - 1.0)
    var_ref[0, 0, 0] = var


def _emb_and_var(x, W, b):
    B, N, T = x.shape
    L = W.shape[0]
    grid = (B, N_NODE_BLKS)
    emb, var = pl.pallas_call(
        _emb_var_body,
        grid=grid,
        in_specs=[
            pl.BlockSpec((1, NODE_BLK, T), lambda i, j: (i, j, 0)),
            pl.BlockSpec((L, T), lambda i, j: (0, 0)),
            pl.BlockSpec((1, L), lambda i, j: (0, 0)),
        ],
        out_specs=[
            pl.BlockSpec((1, NODE_BLK, L), lambda i, j: (i, j, 0)),
            pl.BlockSpec((1, 1, 1, NODE_BLK), lambda i, j: (i, j, 0, 0)),
        ],
        out_shape=[
            jax.ShapeDtypeStruct((B, N, L), jnp.float32),
            jax.ShapeDtypeStruct((B, N_NODE_BLKS, 1, NODE_BLK), jnp.float32),
        ],
    )(x, W, b.reshape(1, L))
    return emb, var.reshape(B, N)


def _aggregate(x, var, level, node_bef):
    sorted_indices = jnp.argsort(-var, axis=-1)
    nc = NUM_CLASSES[level]
    num_per_class = node_bef // nc
    outs = []
    for i in range(nc):
        start_idx = i * num_per_class
        end_idx = (i + 1) * num_per_class if i != nc - 1 else node_bef
        class_indices = sorted_indices[:, start_idx:end_idx]
        gathered = jnp.take_along_axis(
            x[:, :, :TIME_STEPS], class_indices[:, :, None], axis=1)
        outs.append(jnp.mean(gathered, axis=1))
    return jnp.stack(outs, axis=1)


def kernel(x, W, b):
    emb, var1 = _emb_and_var(x, W, b)
    agg1 = _aggregate(emb, var1, 0, NUM_NODES)
    var2 = jnp.var(agg1, axis=2, ddof=1)
    agg2 = _aggregate(agg1, var2, 1, 64)
    B, C, D = agg2.shape
    e = agg2.reshape(B * C, D)
    norms = jnp.sqrt(jnp.sum(e * e, axis=-1))
    dots = e @ e.T
    sim = dots / jnp.maximum(norms[:, None] * norms[None, :], 1e-8)
    logits = sim / TEMPERATURE
    labels = jnp.eye(B * C, dtype=jnp.float32)
    loss = jnp.mean(jnp.maximum(logits, 0.0) - logits * labels
                    + jnp.log1p(jnp.exp(-jnp.abs(logits))))
    return (loss, emb)


# speedup vs baseline: 2.2846x; 2.2846x over previous
import jax
import jax.numpy as jnp
from jax import lax
from jax.experimental import pallas as pl

NUM_NODES = 10000
NODES_PAD = 10240
TIME_STEPS = 64
LATENT_DIM = 256
TEMPERATURE = 0.5
BATCH = 16

NODE_BLK = 1000
N_NODE_BLKS = NUM_NODES // NODE_BLK

NC1 = 64
NPC1 = NUM_NODES // NC1
NC2 = 8
NPC2 = 64 // NC2

_KEY_HI = 0x7F800000


def _emb_var_body(x_ref, w_ref, b_ref, emb_ref, var_ref):
    xb = x_ref[0]
    w = w_ref[...]
    emb = lax.dot_general(xb, w, (((1,), (1,)), ((), ())),
                          preferred_element_type=jnp.float32)
    emb = emb + b_ref[...]
    emb_ref[0] = emb
    s = jnp.sum(emb, axis=1)
    ss = jnp.sum(emb * emb, axis=1)
    n = jnp.float32(LATENT_DIM)
    var_ref[0, 0, 0] = (ss - s * s / n) / (n - 1.0)


def _emb_and_var(x, W, b):
    B, N, T = x.shape
    L = W.shape[0]
    emb, var = pl.pallas_call(
        _emb_var_body,
        grid=(B, N_NODE_BLKS),
        in_specs=[
            pl.BlockSpec((1, NODE_BLK, T), lambda i, j: (i, j, 0)),
            pl.BlockSpec((L, T), lambda i, j: (0, 0)),
            pl.BlockSpec((1, L), lambda i, j: (0, 0)),
        ],
        out_specs=[
            pl.BlockSpec((1, NODE_BLK, L), lambda i, j: (i, j, 0)),
            pl.BlockSpec((1, 1, 1, NODE_BLK), lambda i, j: (i, j, 0, 0)),
        ],
        out_shape=[
            jax.ShapeDtypeStruct((B, N, L), jnp.float32),
            jax.ShapeDtypeStruct((B, N_NODE_BLKS, 1, NODE_BLK), jnp.float32),
        ],
    )(x, W, b.reshape(1, L))
    return emb, var.reshape(B, N)


def _agg_body(var_ref, emb_ref, agg2_ref):
    v = var_ref[0]
    bits = lax.bitcast_convert_type(v, jnp.int32)
    key = jnp.where(bits < 0, bits ^ jnp.int32(0x7FFFFFFF), bits)
    lane = lax.broadcasted_iota(jnp.int32, (1, NODES_PAD), 1)
    key = jnp.where(lane < NUM_NODES, key, jnp.int32(-1065353217))

    r = lax.broadcasted_iota(jnp.int32, (NC1, 1), 0) * NPC1
    lo = jnp.full((NC1, 1), jnp.int32(-1065353218))
    hi = jnp.full((NC1, 1), _KEY_HI, dtype=jnp.int32)

    def step(_, carry):
        lo, hi = carry
        mid = lo + lax.shift_right_logical(hi - lo + 1, 1)
        cnt = jnp.sum((key >= mid).astype(jnp.int32), axis=1, keepdims=True)
        pred = cnt >= r
        return (jnp.where(pred, mid, lo), jnp.where(pred, hi, mid - 1))

    lo, hi = lax.fori_loop(0, 31, step, (lo, hi))
    thr = lo

    in_top = (key >= thr).astype(jnp.int32)
    bucket = (NC1 - 1) - jnp.sum(in_top, axis=0, keepdims=True)
    cls = lax.broadcasted_iota(jnp.int32, (NC1, 1), 0)
    onehot = (bucket == cls).astype(jnp.float32)
    onehot = onehot[:, :NUM_NODES]
    counts = jnp.sum(onehot, axis=1, keepdims=True)
    e64 = emb_ref[0][:, :TIME_STEPS]
    sums = lax.dot_general(onehot, e64, (((1,), (0,)), ((), ())),
                           preferred_element_type=jnp.float32)
    agg1 = sums / jnp.maximum(counts, 1.0)

    s2 = jnp.sum(agg1, axis=1, keepdims=True)
    ss2 = jnp.sum(agg1 * agg1, axis=1, keepdims=True)
    n2 = jnp.float32(TIME_STEPS)
    v2 = (ss2 - s2 * s2 / n2) / (n2 - 1.0)
    eye = (lax.broadcasted_iota(jnp.int32, (NC1, NC1), 0)
           == lax.broadcasted_iota(jnp.int32, (NC1, NC1), 1))
    v2r = jnp.sum(jnp.where(eye, v2, 0.0), axis=0, keepdims=True)
    irow = lax.broadcasted_iota(jnp.int32, (NC1, NC1), 0)
    jcol = lax.broadcasted_iota(jnp.int32, (NC1, NC1), 1)
    ahead = (v2 > v2r) | ((v2 == v2r) & (irow < jcol))
    rank2 = jnp.sum(ahead.astype(jnp.int32), axis=0, keepdims=True)
    bucket2 = rank2 // NPC2
    cls2 = lax.broadcasted_iota(jnp.int32, (NC2, 1), 0)
    onehot2 = (bucket2 == cls2).astype(jnp.float32)
    agg2 = lax.dot_general(onehot2, agg1, (((1,), (0,)), ((), ())),
                           preferred_element_type=jnp.float32)
    agg2_ref[0] = agg2 * jnp.float32(1.0 / NPC2)


def _aggregate2(var_pad, emb):
    B = emb.shape[0]
    return pl.pallas_call(
        _agg_body,
        grid=(B,),
        in_specs=[
            pl.BlockSpec((1, 1, NODES_PAD), lambda i: (i, 0, 0)),
            pl.BlockSpec((1, NUM_NODES, 128), lambda i: (i, 0, 0)),
        ],
        out_specs=pl.BlockSpec((1, NC2, TIME_STEPS), lambda i: (i, 0, 0)),
        out_shape=jax.ShapeDtypeStruct((B, NC2, TIME_STEPS), jnp.float32),
    )(var_pad, emb)


def _loss_body(agg2_ref, loss_ref):
    e = agg2_ref[...].reshape(BATCH * NC2, TIME_STEPS)
    nsq = jnp.sum(e * e, axis=1, keepdims=True)
    n1 = jnp.sqrt(nsq)
    m = BATCH * NC2
    eye = (lax.broadcasted_iota(jnp.int32, (m, m), 0)
           == lax.broadcasted_iota(jnp.int32, (m, m), 1))
    n1r = jnp.sum(jnp.where(eye, n1, 0.0), axis=0, keepdims=True)
    dots = lax.dot_general(e, e, (((1,), (1,)), ((), ())),
                           preferred_element_type=jnp.float32)
    sim = dots / jnp.maximum(n1 * n1r, 1e-8)
    logits = sim * jnp.float32(1.0 / TEMPERATURE)
    lab = eye.astype(jnp.float32)
    loss_mat = (jnp.maximum(logits, 0.0) - logits * lab
                + jnp.log1p(jnp.exp(-jnp.abs(logits))))
    tot = jnp.sum(jnp.sum(loss_mat, axis=1, keepdims=True),
                  axis=0, keepdims=True)
    loss_ref[...] = tot * jnp.float32(1.0 / (m * m))


def _loss(agg2):
    out = pl.pallas_call(
        _loss_body,
        out_shape=jax.ShapeDtypeStruct((1, 1), jnp.float32),
    )(agg2)
    return out.reshape(())


def kernel(x, W, b):
    emb, var1 = _emb_and_var(x, W, b)
    var_pad = jnp.pad(var1, ((0, 0), (0, NODES_PAD - NUM_NODES)),
                      constant_values=-1.0).reshape(BATCH, 1, NODES_PAD)
    agg2 = _aggregate2(var_pad, emb)
    return (_loss(agg2), emb)


# speedup vs baseline: 2.5692x; 1.1245x over previous
import jax
import jax.numpy as jnp
from jax import lax
from jax.experimental import pallas as pl
from jax.experimental.pallas import tpu as pltpu

NUM_NODES = 10000
NODES_PAD = 10240
TIME_STEPS = 64
LATENT_DIM = 256
TEMPERATURE = 0.5
BATCH = 16

NODE_BLK = 2000
N_NODE_BLKS = NUM_NODES // NODE_BLK

NC1 = 64
NPC1 = NUM_NODES // NC1
NC2 = 8
NPC2 = 64 // NC2

_KEY_HI = 0x7F800000


def _emb_var_body(x_ref, w_ref, b_ref, emb_ref, var_ref):
    xb = x_ref[0]
    w = w_ref[...]
    emb = lax.dot_general(xb, w, (((1,), (1,)), ((), ())),
                          preferred_element_type=jnp.float32)
    emb = emb + b_ref[...]
    emb_ref[0] = emb
    s = jnp.sum(emb, axis=1)
    ss = jnp.sum(emb * emb, axis=1)
    n = jnp.float32(LATENT_DIM)
    var_ref[0, 0, 0] = (ss - s * s / n) / (n - 1.0)


def _emb_and_var(x, W, b):
    B, N, T = x.shape
    L = W.shape[0]
    emb, var = pl.pallas_call(
        _emb_var_body,
        grid=(B, N_NODE_BLKS),
        in_specs=[
            pl.BlockSpec((1, NODE_BLK, T), lambda i, j: (i, j, 0)),
            pl.BlockSpec((L, T), lambda i, j: (0, 0)),
            pl.BlockSpec((1, L), lambda i, j: (0, 0)),
        ],
        out_specs=[
            pl.BlockSpec((1, NODE_BLK, L), lambda i, j: (i, j, 0)),
            pl.BlockSpec((1, 1, 1, NODE_BLK), lambda i, j: (i, j, 0, 0)),
        ],
        out_shape=[
            jax.ShapeDtypeStruct((B, N, L), jnp.float32),
            jax.ShapeDtypeStruct((B, N_NODE_BLKS, 1, NODE_BLK), jnp.float32),
        ],
        compiler_params=pltpu.CompilerParams(
            dimension_semantics=("parallel", "parallel")),
    )(x, W, b.reshape(1, L))
    return emb, var.reshape(B, N)


def _agg_body(var_ref, emb_ref, agg2_ref):
    v = var_ref[0]
    bits = lax.bitcast_convert_type(v, jnp.int32)
    key = jnp.where(bits < 0, bits ^ jnp.int32(0x7FFFFFFF), bits)
    lane = lax.broadcasted_iota(jnp.int32, (1, NODES_PAD), 1)
    key = jnp.where(lane < NUM_NODES, key, jnp.int32(-1065353217))

    r = lax.broadcasted_iota(jnp.int32, (NC1, 1), 0) * NPC1
    lo = jnp.full((NC1, 1), jnp.int32(-1065353218))
    hi = jnp.full((NC1, 1), _KEY_HI, dtype=jnp.int32)

    def step(_, carry):
        lo, hi = carry
        mid = lo + lax.shift_right_logical(hi - lo + 1, 1)
        cnt = jnp.sum((key >= mid).astype(jnp.int32), axis=1, keepdims=True)
        pred = cnt >= r
        return (jnp.where(pred, mid, lo), jnp.where(pred, hi, mid - 1))

    lo, hi = lax.fori_loop(0, 31, step, (lo, hi))
    thr = lo

    in_top = (key >= thr).astype(jnp.int32)
    bucket = (NC1 - 1) - jnp.sum(in_top, axis=0, keepdims=True)
    cls = lax.broadcasted_iota(jnp.int32, (NC1, 1), 0)
    onehot = (bucket == cls).astype(jnp.float32)
    onehot = onehot[:, :NUM_NODES]
    counts = jnp.sum(onehot, axis=1, keepdims=True)
    e64 = emb_ref[0][:, :TIME_STEPS]
    sums = lax.dot_general(onehot, e64, (((1,), (0,)), ((), ())),
                           preferred_element_type=jnp.float32)
    agg1 = sums / jnp.maximum(counts, 1.0)

    s2 = jnp.sum(agg1, axis=1, keepdims=True)
    ss2 = jnp.sum(agg1 * agg1, axis=1, keepdims=True)
    n2 = jnp.float32(TIME_STEPS)
    v2 = (ss2 - s2 * s2 / n2) / (n2 - 1.0)
    eye = (lax.broadcasted_iota(jnp.int32, (NC1, NC1), 0)
           == lax.broadcasted_iota(jnp.int32, (NC1, NC1), 1))
    v2r = jnp.sum(jnp.where(eye, v2, 0.0), axis=0, keepdims=True)
    irow = lax.broadcasted_iota(jnp.int32, (NC1, NC1), 0)
    jcol = lax.broadcasted_iota(jnp.int32, (NC1, NC1), 1)
    ahead = (v2 > v2r) | ((v2 == v2r) & (irow < jcol))
    rank2 = jnp.sum(ahead.astype(jnp.int32), axis=0, keepdims=True)
    bucket2 = rank2 // NPC2
    cls2 = lax.broadcasted_iota(jnp.int32, (NC2, 1), 0)
    onehot2 = (bucket2 == cls2).astype(jnp.float32)
    agg2 = lax.dot_general(onehot2, agg1, (((1,), (0,)), ((), ())),
                           preferred_element_type=jnp.float32)
    agg2_ref[0] = agg2 * jnp.float32(1.0 / NPC2)


def _aggregate2(var_pad, emb):
    B = emb.shape[0]
    return pl.pallas_call(
        _agg_body,
        grid=(B,),
        in_specs=[
            pl.BlockSpec((1, 1, NODES_PAD), lambda i: (i, 0, 0)),
            pl.BlockSpec((1, NUM_NODES, 128), lambda i: (i, 0, 0)),
        ],
        out_specs=pl.BlockSpec((1, NC2, TIME_STEPS), lambda i: (i, 0, 0)),
        out_shape=jax.ShapeDtypeStruct((B, NC2, TIME_STEPS), jnp.float32),
    )(var_pad, emb)


def _loss_body(agg2_ref, loss_ref):
    e = agg2_ref[...].reshape(BATCH * NC2, TIME_STEPS)
    nsq = jnp.sum(e * e, axis=1, keepdims=True)
    n1 = jnp.sqrt(nsq)
    m = BATCH * NC2
    eye = (lax.broadcasted_iota(jnp.int32, (m, m), 0)
           == lax.broadcasted_iota(jnp.int32, (m, m), 1))
    n1r = jnp.sum(jnp.where(eye, n1, 0.0), axis=0, keepdims=True)
    dots = lax.dot_general(e, e, (((1,), (1,)), ((), ())),
                           preferred_element_type=jnp.float32)
    sim = dots / jnp.maximum(n1 * n1r, 1e-8)
    logits = sim * jnp.float32(1.0 / TEMPERATURE)
    lab = eye.astype(jnp.float32)
    loss_mat = (jnp.maximum(logits, 0.0) - logits * lab
                + jnp.log1p(jnp.exp(-jnp.abs(logits))))
    tot = jnp.sum(jnp.sum(loss_mat, axis=1, keepdims=True),
                  axis=0, keepdims=True)
    loss_ref[...] = tot * jnp.float32(1.0 / (m * m))


def _loss(agg2):
    out = pl.pallas_call(
        _loss_body,
        out_shape=jax.ShapeDtypeStruct((1, 1), jnp.float32),
    )(agg2)
    return out.reshape(())


def kernel(x, W, b):
    emb, var1 = _emb_and_var(x, W, b)
    var_pad = jnp.pad(var1, ((0, 0), (0, NODES_PAD - NUM_NODES)),
                      constant_values=-1.0).reshape(BATCH, 1, NODES_PAD)
    agg2 = _aggregate2(var_pad, emb)
    return (_loss(agg2), emb)


# speedup vs baseline: 2.6138x; 1.0174x over previous
import jax
import jax.numpy as jnp
from jax import lax
from jax.experimental import pallas as pl
from jax.experimental.pallas import tpu as pltpu

NUM_NODES = 10000
NODES_PAD = 10240
TIME_STEPS = 64
LATENT_DIM = 256
TEMPERATURE = 0.5
BATCH = 16

NODE_BLK = 5000
N_NODE_BLKS = NUM_NODES // NODE_BLK

NC1 = 64
NPC1 = NUM_NODES // NC1
NC2 = 8
NPC2 = 64 // NC2

_KEY_HI = 0x7F800000


def _emb_var_body(x_ref, w_ref, b_ref, emb_ref, var_ref):
    xb = x_ref[0]
    w = w_ref[...]
    emb = lax.dot_general(xb, w, (((1,), (1,)), ((), ())),
                          preferred_element_type=jnp.float32)
    emb = emb + b_ref[...]
    emb_ref[0] = emb
    s = jnp.sum(emb, axis=1)
    ss = jnp.sum(emb * emb, axis=1)
    n = jnp.float32(LATENT_DIM)
    var_ref[0, 0, 0] = (ss - s * s / n) / (n - 1.0)


def _emb_and_var(x, W, b):
    B, N, T = x.shape
    L = W.shape[0]
    emb, var = pl.pallas_call(
        _emb_var_body,
        grid=(B, N_NODE_BLKS),
        in_specs=[
            pl.BlockSpec((1, NODE_BLK, T), lambda i, j: (i, j, 0)),
            pl.BlockSpec((L, T), lambda i, j: (0, 0)),
            pl.BlockSpec((1, L), lambda i, j: (0, 0)),
        ],
        out_specs=[
            pl.BlockSpec((1, NODE_BLK, L), lambda i, j: (i, j, 0)),
            pl.BlockSpec((1, 1, 1, NODE_BLK), lambda i, j: (i, j, 0, 0)),
        ],
        out_shape=[
            jax.ShapeDtypeStruct((B, N, L), jnp.float32),
            jax.ShapeDtypeStruct((B, N_NODE_BLKS, 1, NODE_BLK), jnp.float32),
        ],
        compiler_params=pltpu.CompilerParams(
            dimension_semantics=("parallel", "parallel")),
    )(x, W, b.reshape(1, L))
    return emb, var.reshape(B, N)


def _agg_body(var_ref, emb_ref, agg2_ref):
    v = var_ref[0]
    bits = lax.bitcast_convert_type(v, jnp.int32)
    key = jnp.where(bits < 0, bits ^ jnp.int32(0x7FFFFFFF), bits)
    lane = lax.broadcasted_iota(jnp.int32, (1, NODES_PAD), 1)
    key = jnp.where(lane < NUM_NODES, key, jnp.int32(-1065353217))

    r = lax.broadcasted_iota(jnp.int32, (NC1, 1), 0) * NPC1
    lo = jnp.full((NC1, 1), jnp.int32(-1065353218))
    hi = jnp.full((NC1, 1), _KEY_HI, dtype=jnp.int32)

    def step(_, carry):
        lo, hi = carry
        mid = lo + lax.shift_right_logical(hi - lo + 1, 1)
        cnt = jnp.sum((key >= mid).astype(jnp.int32), axis=1, keepdims=True)
        pred = cnt >= r
        return (jnp.where(pred, mid, lo), jnp.where(pred, hi, mid - 1))

    lo, hi = lax.fori_loop(0, 31, step, (lo, hi))
    thr = lo

    in_top = (key >= thr).astype(jnp.int32)
    bucket = (NC1 - 1) - jnp.sum(in_top, axis=0, keepdims=True)
    cls = lax.broadcasted_iota(jnp.int32, (NC1, 1), 0)
    onehot = (bucket == cls).astype(jnp.float32)
    onehot = onehot[:, :NUM_NODES]
    counts = jnp.sum(onehot, axis=1, keepdims=True)
    e64 = emb_ref[0][:, :TIME_STEPS]
    sums = lax.dot_general(onehot, e64, (((1,), (0,)), ((), ())),
                           preferred_element_type=jnp.float32)
    agg1 = sums / jnp.maximum(counts, 1.0)

    s2 = jnp.sum(agg1, axis=1, keepdims=True)
    ss2 = jnp.sum(agg1 * agg1, axis=1, keepdims=True)
    n2 = jnp.float32(TIME_STEPS)
    v2 = (ss2 - s2 * s2 / n2) / (n2 - 1.0)
    eye = (lax.broadcasted_iota(jnp.int32, (NC1, NC1), 0)
           == lax.broadcasted_iota(jnp.int32, (NC1, NC1), 1))
    v2r = jnp.sum(jnp.where(eye, v2, 0.0), axis=0, keepdims=True)
    irow = lax.broadcasted_iota(jnp.int32, (NC1, NC1), 0)
    jcol = lax.broadcasted_iota(jnp.int32, (NC1, NC1), 1)
    ahead = (v2 > v2r) | ((v2 == v2r) & (irow < jcol))
    rank2 = jnp.sum(ahead.astype(jnp.int32), axis=0, keepdims=True)
    bucket2 = rank2 // NPC2
    cls2 = lax.broadcasted_iota(jnp.int32, (NC2, 1), 0)
    onehot2 = (bucket2 == cls2).astype(jnp.float32)
    agg2 = lax.dot_general(onehot2, agg1, (((1,), (0,)), ((), ())),
                           preferred_element_type=jnp.float32)
    agg2_ref[0] = agg2 * jnp.float32(1.0 / NPC2)


def _aggregate2(var_pad, emb):
    B = emb.shape[0]
    return pl.pallas_call(
        _agg_body,
        grid=(B,),
        in_specs=[
            pl.BlockSpec((1, 1, NODES_PAD), lambda i: (i, 0, 0)),
            pl.BlockSpec((1, NUM_NODES, 128), lambda i: (i, 0, 0)),
        ],
        out_specs=pl.BlockSpec((1, NC2, TIME_STEPS), lambda i: (i, 0, 0)),
        out_shape=jax.ShapeDtypeStruct((B, NC2, TIME_STEPS), jnp.float32),
    )(var_pad, emb)


def _loss_body(agg2_ref, loss_ref):
    e = agg2_ref[...].reshape(BATCH * NC2, TIME_STEPS)
    nsq = jnp.sum(e * e, axis=1, keepdims=True)
    n1 = jnp.sqrt(nsq)
    m = BATCH * NC2
    eye = (lax.broadcasted_iota(jnp.int32, (m, m), 0)
           == lax.broadcasted_iota(jnp.int32, (m, m), 1))
    n1r = jnp.sum(jnp.where(eye, n1, 0.0), axis=0, keepdims=True)
    dots = lax.dot_general(e, e, (((1,), (1,)), ((), ())),
                           preferred_element_type=jnp.float32)
    sim = dots / jnp.maximum(n1 * n1r, 1e-8)
    logits = sim * jnp.float32(1.0 / TEMPERATURE)
    lab = eye.astype(jnp.float32)
    loss_mat = (jnp.maximum(logits, 0.0) - logits * lab
                + jnp.log1p(jnp.exp(-jnp.abs(logits))))
    tot = jnp.sum(jnp.sum(loss_mat, axis=1, keepdims=True),
                  axis=0, keepdims=True)
    loss_ref[...] = tot * jnp.float32(1.0 / (m * m))


def _loss(agg2):
    out = pl.pallas_call(
        _loss_body,
        out_shape=jax.ShapeDtypeStruct((1, 1), jnp.float32),
    )(agg2)
    return out.reshape(())


def kernel(x, W, b):
    emb, var1 = _emb_and_var(x, W, b)
    var_pad = jnp.pad(var1, ((0, 0), (0, NODES_PAD - NUM_NODES)),
                      constant_values=-1.0).reshape(BATCH, 1, NODES_PAD)
    agg2 = _aggregate2(var_pad, emb)
    return (_loss(agg2), emb)


# speedup vs baseline: 3.5570x; 1.3609x over previous
import jax
import jax.numpy as jnp
from jax import lax
from jax.experimental import pallas as pl
from jax.experimental.pallas import tpu as pltpu

NUM_NODES = 10000
NODES_PAD = 10240
TIME_STEPS = 64
LATENT_DIM = 256
TEMPERATURE = 0.5
BATCH = 16

NODE_BLK = 5000
N_NODE_BLKS = NUM_NODES // NODE_BLK

NC1 = 64
NPC1 = NUM_NODES // NC1
NC2 = 8
NPC2 = 64 // NC2

_KEY_HI = 0x7F800000


def _emb_var_body(x_ref, w_ref, b_ref, emb_ref, var_ref):
    xb = x_ref[0]
    w = w_ref[...]
    emb = lax.dot_general(xb, w, (((1,), (1,)), ((), ())),
                          preferred_element_type=jnp.float32)
    emb = emb + b_ref[...]
    emb_ref[0] = emb
    s = jnp.sum(emb, axis=1)
    ss = jnp.sum(emb * emb, axis=1)
    n = jnp.float32(LATENT_DIM)
    var_ref[0, 0, 0] = (ss - s * s / n) / (n - 1.0)


def _emb_and_var(x, W, b):
    B, N, T = x.shape
    L = W.shape[0]
    emb, var = pl.pallas_call(
        _emb_var_body,
        grid=(B, N_NODE_BLKS),
        in_specs=[
            pl.BlockSpec((1, NODE_BLK, T), lambda i, j: (i, j, 0)),
            pl.BlockSpec((L, T), lambda i, j: (0, 0)),
            pl.BlockSpec((1, L), lambda i, j: (0, 0)),
        ],
        out_specs=[
            pl.BlockSpec((1, NODE_BLK, L), lambda i, j: (i, j, 0)),
            pl.BlockSpec((1, 1, 1, NODE_BLK), lambda i, j: (i, j, 0, 0)),
        ],
        out_shape=[
            jax.ShapeDtypeStruct((B, N, L), jnp.float32),
            jax.ShapeDtypeStruct((B, N_NODE_BLKS, 1, NODE_BLK), jnp.float32),
        ],
        compiler_params=pltpu.CompilerParams(
            dimension_semantics=("parallel", "parallel")),
    )(x, W, b.reshape(1, L))
    return emb, var.reshape(B, N)


def _agg_body(var_ref, emb_ref, agg2_ref):
    v = var_ref[0]
    bits = lax.bitcast_convert_type(v, jnp.int32)
    mono = jnp.where(bits < 0, bits ^ jnp.int32(0x7FFFFFFF), bits)
    key = jnp.maximum(mono, jnp.int32(-1)) + 1
    lane = lax.broadcasted_iota(jnp.int32, (1, NODES_PAD), 1)
    key = jnp.where(lane < NUM_NODES, key, 0)

    r = lax.broadcasted_iota(jnp.int32, (NC1, 1), 0) * NPC1
    rf = r.astype(jnp.float32)
    dcls = lax.broadcasted_iota(jnp.int32, (NC1, 1), 0)
    drow = lax.broadcasted_iota(jnp.int32, (NC1, NC1), 1)
    utri = (lax.broadcasted_iota(jnp.int32, (NC1, NC1), 0)
            >= drow).astype(jnp.float32)
    P = jnp.zeros((NC1, 1), jnp.int32)
    a = jnp.zeros((NC1, 1), jnp.float32)
    for s in (30, 24, 18, 12, 6, 0):
        pref = lax.shift_right_logical(key, min(s + 6, 31))
        cand = (pref == P).astype(jnp.float32)
        dig = lax.shift_right_logical(key, s) & 63
        donehot = (dig == dcls).astype(jnp.float32)
        c = lax.dot_general(cand, donehot, (((1,), (1,)), ((), ())),
                            preferred_element_type=jnp.float32)
        st = lax.dot_general(c, utri, (((1,), (0,)), ((), ())),
                             preferred_element_type=jnp.float32)
        t = a + st
        ok = (t >= rf).astype(jnp.int32)
        dstar = jnp.sum(ok, axis=1, keepdims=True) - 1
        sel = (drow == dstar).astype(jnp.float32)
        a = jnp.sum((t - c) * sel, axis=1, keepdims=True)
        P = P * 64 + dstar
    thr = jnp.where(r >= 1, P, jnp.int32(0x7FFFFFFF))

    in_top = (key >= thr).astype(jnp.int32)
    bucket = (NC1 - 1) - jnp.sum(in_top, axis=0, keepdims=True)
    cls = lax.broadcasted_iota(jnp.int32, (NC1, 1), 0)
    onehot = (bucket == cls).astype(jnp.float32)
    onehot = onehot[:, :NUM_NODES]
    counts = jnp.sum(onehot, axis=1, keepdims=True)
    e64 = emb_ref[0][:, :TIME_STEPS]
    sums = lax.dot_general(onehot, e64, (((1,), (0,)), ((), ())),
                           preferred_element_type=jnp.float32)
    agg1 = sums / jnp.maximum(counts, 1.0)

    s2 = jnp.sum(agg1, axis=1, keepdims=True)
    ss2 = jnp.sum(agg1 * agg1, axis=1, keepdims=True)
    n2 = jnp.float32(TIME_STEPS)
    v2 = (ss2 - s2 * s2 / n2) / (n2 - 1.0)
    eye = (lax.broadcasted_iota(jnp.int32, (NC1, NC1), 0)
           == lax.broadcasted_iota(jnp.int32, (NC1, NC1), 1))
    v2r = jnp.sum(jnp.where(eye, v2, 0.0), axis=0, keepdims=True)
    irow = lax.broadcasted_iota(jnp.int32, (NC1, NC1), 0)
    jcol = lax.broadcasted_iota(jnp.int32, (NC1, NC1), 1)
    ahead = (v2 > v2r) | ((v2 == v2r) & (irow < jcol))
    rank2 = jnp.sum(ahead.astype(jnp.int32), axis=0, keepdims=True)
    bucket2 = rank2 // NPC2
    cls2 = lax.broadcasted_iota(jnp.int32, (NC2, 1), 0)
    onehot2 = (bucket2 == cls2).astype(jnp.float32)
    agg2 = lax.dot_general(onehot2, agg1, (((1,), (0,)), ((), ())),
                           preferred_element_type=jnp.float32)
    agg2_ref[0] = agg2 * jnp.float32(1.0 / NPC2)


def _aggregate2(var_pad, emb):
    B = emb.shape[0]
    return pl.pallas_call(
        _agg_body,
        grid=(B,),
        in_specs=[
            pl.BlockSpec((1, 1, NODES_PAD), lambda i: (i, 0, 0)),
            pl.BlockSpec((1, NUM_NODES, 128), lambda i: (i, 0, 0)),
        ],
        out_specs=pl.BlockSpec((1, NC2, TIME_STEPS), lambda i: (i, 0, 0)),
        out_shape=jax.ShapeDtypeStruct((B, NC2, TIME_STEPS), jnp.float32),
    )(var_pad, emb)


def _loss_body(agg2_ref, loss_ref):
    e = agg2_ref[...].reshape(BATCH * NC2, TIME_STEPS)
    nsq = jnp.sum(e * e, axis=1, keepdims=True)
    n1 = jnp.sqrt(nsq)
    m = BATCH * NC2
    eye = (lax.broadcasted_iota(jnp.int32, (m, m), 0)
           == lax.broadcasted_iota(jnp.int32, (m, m), 1))
    n1r = jnp.sum(jnp.where(eye, n1, 0.0), axis=0, keepdims=True)
    dots = lax.dot_general(e, e, (((1,), (1,)), ((), ())),
                           preferred_element_type=jnp.float32)
    sim = dots / jnp.maximum(n1 * n1r, 1e-8)
    logits = sim * jnp.float32(1.0 / TEMPERATURE)
    lab = eye.astype(jnp.float32)
    loss_mat = (jnp.maximum(logits, 0.0) - logits * lab
                + jnp.log1p(jnp.exp(-jnp.abs(logits))))
    tot = jnp.sum(jnp.sum(loss_mat, axis=1, keepdims=True),
                  axis=0, keepdims=True)
    loss_ref[...] = tot * jnp.float32(1.0 / (m * m))


def _loss(agg2):
    out = pl.pallas_call(
        _loss_body,
        out_shape=jax.ShapeDtypeStruct((1, 1), jnp.float32),
    )(agg2)
    return out.reshape(())


def kernel(x, W, b):
    emb, var1 = _emb_and_var(x, W, b)
    var_pad = jnp.pad(var1, ((0, 0), (0, NODES_PAD - NUM_NODES)),
                      constant_values=-1.0).reshape(BATCH, 1, NODES_PAD)
    agg2 = _aggregate2(var_pad, emb)
    return (_loss(agg2), emb)
